# Initial kernel scaffold; baseline (speedup 1.0000x reference)
import jax
import jax.numpy as jnp
from jax.experimental import pallas as pl
from jax.experimental.pallas import tpu as pltpu
from jax.experimental.pallas import tpu_sc as plsc

_N = 10000
_E = 160000
_EPAD = 163840
_DD = 128
_EH = 256
_TW = 400
_GW = 64
_SW = 128
_ET = 2048
_NT = 1000
_NSUB = 16
_ROWS_PER_SUB = _N // _NSUB

_F32 = jnp.float32


def _vmesh():
    return plsc.VectorSubcoreMesh(core_axis_name="core", subcore_axis_name="subcore")


def _pre_body(xs_ref, w1a_ref, w1b_ref, a_ref, b_ref):
    xs = xs_ref[...]
    a_ref[...] = jnp.dot(xs, w1a_ref[...], preferred_element_type=_F32)
    b_ref[...] = jnp.dot(xs, w1b_ref[...], preferred_element_type=_F32)


def _precompute(x_s, w1a, w1b):
    return pl.pallas_call(
        _pre_body,
        grid=(_N // _NT,),
        in_specs=[
            pl.BlockSpec((_NT, _DD), lambda i: (i, 0)),
            pl.BlockSpec((_DD, _EH), lambda i: (0, 0)),
            pl.BlockSpec((_DD, _EH), lambda i: (0, 0)),
        ],
        out_specs=[
            pl.BlockSpec((_NT, _EH), lambda i: (i, 0)),
            pl.BlockSpec((_NT, _EH), lambda i: (i, 0)),
        ],
        out_shape=[
            jax.ShapeDtypeStruct((_N, _EH), _F32),
            jax.ShapeDtypeStruct((_N, _EH), _F32),
        ],
    )(x_s, w1a, w1b)


def _tables_body(out_ref, p_ref, fk_ref, a_ref, b_ref, w1c_ref, w1d_ref,
                 onew_ref, ts_ref, td_ref):
    out_new = out_ref[...] + jnp.dot(p_ref[0] + p_ref[1], fk_ref[...],
                                     preferred_element_type=_F32)
    onew_ref[...] = out_new
    m = (jnp.sum(out_new, axis=1, keepdims=True) != 0.0).astype(_F32)
    mcol = jnp.broadcast_to(m, (m.shape[0], 16))
    ts_ref[:, 0:256] = a_ref[...] + jnp.dot(out_new, w1c_ref[...],
                                            preferred_element_type=_F32)
    ts_ref[:, 256:384] = out_new
    ts_ref[:, 384:400] = mcol
    td_ref[:, 0:256] = b_ref[...] + jnp.dot(out_new, w1d_ref[...],
                                            preferred_element_type=_F32)
    td_ref[:, 256:384] = out_new
    td_ref[:, 384:400] = mcol


def _tables(out_prev, partials, fk, a_tab, b_tab, w1c, w1d):
    return pl.pallas_call(
        _tables_body,
        grid=(_N // _NT,),
        in_specs=[
            pl.BlockSpec((_NT, _DD), lambda i: (i, 0)),
            pl.BlockSpec((2, _NT, _DD), lambda i: (0, i, 0)),
            pl.BlockSpec((_DD, _DD), lambda i: (0, 0)),
            pl.BlockSpec((_NT, _EH), lambda i: (i, 0)),
            pl.BlockSpec((_NT, _EH), lambda i: (i, 0)),
            pl.BlockSpec((_DD, _EH), lambda i: (0, 0)),
            pl.BlockSpec((_DD, _EH), lambda i: (0, 0)),
        ],
        out_specs=[
            pl.BlockSpec((_NT, _DD), lambda i: (i, 0)),
            pl.BlockSpec((_NT, _TW), lambda i: (i, 0)),
            pl.BlockSpec((_NT, _TW), lambda i: (i, 0)),
        ],
        out_shape=[
            jax.ShapeDtypeStruct((_N, _DD), _F32),
            jax.ShapeDtypeStruct((_N, _TW), _F32),
            jax.ShapeDtypeStruct((_N, _TW), _F32),
        ],
    )(out_prev, partials, fk, a_tab, b_tab, w1c, w1d)


def _mlp_body(gs_ref, gd_ref, ea_ref, w1e_ref, b1_ref, w2_ref, b2_ref, a_ref,
              shift_ref):
    gs = gs_ref[...]
    gd = gd_ref[...]
    eb = jnp.dot(ea_ref[...], w1e_ref[...], preferred_element_type=_F32)
    h = gs[:, 0:256] + gd[:, 0:256] + (eb + b1_ref[...])
    alpha = a_ref[0, 0]
    h = jnp.where(h > 0.0, h, alpha * h)
    w = jnp.dot(h, w2_ref[...], preferred_element_type=_F32) + b2_ref[...]
    w = w / jnp.sqrt(jnp.sum(w * w, axis=1, keepdims=True))
    w = jnp.where(jnp.isnan(w), 0.0, w)
    em = jnp.max(jnp.maximum(gs[:, 384:400], gd[:, 384:400]), axis=1,
                 keepdims=True)
    shift_ref[...] = (gd[:, 256:384] - gs[:, 256:384]) * w * em


def _mlp(gsrc, gdst, ea_pad, w1e, b1r, w2, b2r, ar):
    return pl.pallas_call(
        _mlp_body,
        grid=(_EPAD // _ET,),
        in_specs=[
            pl.BlockSpec((_ET, _TW), lambda i: (i, 0)),
            pl.BlockSpec((_ET, _TW), lambda i: (i, 0)),
            pl.BlockSpec((_ET, 16), lambda i: (i, 0)),
            pl.BlockSpec((16, _EH), lambda i: (0, 0)),
            pl.BlockSpec((1, _EH), lambda i: (0, 0)),
            pl.BlockSpec((_EH, _DD), lambda i: (0, 0)),
            pl.BlockSpec((1, _DD), lambda i: (0, 0)),
            pl.BlockSpec((1, 1), lambda i: (0, 0)),
        ],
        out_specs=pl.BlockSpec((_ET, _DD), lambda i: (i, 0)),
        out_shape=jax.ShapeDtypeStruct((_EPAD, _DD), _F32),
    )(gsrc, gdst, ea_pad, w1e, b1r, w2, b2r, ar)


def _final_body(out_ref, p_ref, fk_ref, onew_ref):
    onew_ref[...] = out_ref[...] + jnp.dot(p_ref[0] + p_ref[1], fk_ref[...],
                                           preferred_element_type=_F32)


def _final(out_prev, partials, fk):
    return pl.pallas_call(
        _final_body,
        grid=(_N // _NT,),
        in_specs=[
            pl.BlockSpec((_NT, _DD), lambda i: (i, 0)),
            pl.BlockSpec((2, _NT, _DD), lambda i: (0, i, 0)),
            pl.BlockSpec((_DD, _DD), lambda i: (0, 0)),
        ],
        out_specs=pl.BlockSpec((_NT, _DD), lambda i: (i, 0)),
        out_shape=jax.ShapeDtypeStruct((_N, _DD), _F32),
    )(out_prev, partials, fk)


def _sc_gather(tsrc, tdst, src_idx, dst_idx):
    @pl.kernel(
        out_type=(
            jax.ShapeDtypeStruct((_EPAD, _TW), _F32),
            jax.ShapeDtypeStruct((_EPAD, _TW), _F32),
        ),
        mesh=_vmesh(),
    )
    def k(ts_hbm, td_hbm, si_hbm, di_hbm, gs_hbm, gd_hbm):
        def body(si_v, di_v, gs_v, gd_v):
            pltpu.sync_copy(ts_hbm.at[si_v.at[0]], gs_v)
            pltpu.sync_copy(td_hbm.at[di_v.at[0]], gd_v)

        pltpu.emit_pipeline(
            body,
            grid=(_EPAD // _GW,),
            in_specs=[
                pl.BlockSpec((1, _GW), lambda i: (0, i)),
                pl.BlockSpec((1, _GW), lambda i: (0, i)),
            ],
            out_specs=[
                pl.BlockSpec((_GW, _TW), lambda i: (i, 0)),
                pl.BlockSpec((_GW, _TW), lambda i: (i, 0)),
            ],
            core_axis_name=("core", "subcore"),
            dimension_semantics=(pltpu.PARALLEL,),
        )(si_hbm, di_hbm, gs_hbm, gd_hbm)

    return k(tsrc, tdst, src_idx, dst_idx)


def _sc_scatter(shift, dst_idx, zeros_nd):
    @pl.kernel(
        out_type=jax.ShapeDtypeStruct((2, _N, _DD), _F32),
        mesh=_vmesh(),
        scratch_types=[pltpu.VMEM_SHARED((_N, _DD), _F32)],
    )
    def k(shift_hbm, di_hbm, z_hbm, part_hbm, acc):
        core = jax.lax.axis_index("core")
        sid = jax.lax.axis_index("subcore")
        row0 = sid * _ROWS_PER_SUB
        pltpu.sync_copy(z_hbm.at[pl.ds(row0, _ROWS_PER_SUB)],
                        acc.at[pl.ds(row0, _ROWS_PER_SUB)])
        plsc.subcore_barrier()

        def body(sh_v, di_v):
            pltpu.sync_copy(sh_v, acc.at[di_v.at[0]], add=True)

        pltpu.emit_pipeline(
            body,
            grid=(_EPAD // _SW,),
            in_specs=[
                pl.BlockSpec((_SW, _DD), lambda i: (i, 0)),
                pl.BlockSpec((1, _SW), lambda i: (0, i)),
            ],
            out_specs=[],
            core_axis_name=("core", "subcore"),
            dimension_semantics=(pltpu.PARALLEL,),
        )(shift_hbm, di_hbm)
        plsc.subcore_barrier()
        pltpu.sync_copy(acc.at[pl.ds(row0, _ROWS_PER_SUB)],
                        part_hbm.at[core, pl.ds(row0, _ROWS_PER_SUB)])

    return k(shift, dst_idx, zeros_nd)


def kernel(x_s, x_t, edge_index, edge_attr, W1, b1, a, W2, b2, F):
    w1a = W1[0:128]
    w1b = W1[128:256]
    w1c = W1[256:384]
    w1d = W1[384:512]
    w1e = W1[512:528]
    b1r = b1.reshape(1, _EH)
    b2r = b2.reshape(1, _DD)
    ar = jnp.reshape(a, (1, 1))

    pad = _EPAD - _E
    src_idx = jnp.pad(edge_index[0], (0, pad)).reshape(1, _EPAD)
    dst_idx = jnp.pad(edge_index[1], (0, pad)).reshape(1, _EPAD)
    ea_pad = jnp.pad(edge_attr, ((0, pad), (0, 0)))
    zeros_nd = jnp.zeros((_N, _DD), _F32)

    a_tab, b_tab = _precompute(x_s, w1a, w1b)

    out = jnp.zeros((_N, _DD), _F32)
    partials = jnp.concatenate(
        [x_t.reshape(1, _N, _DD), jnp.zeros((1, _N, _DD), _F32)], axis=0)

    for k in range(4):
        out, tsrc, tdst = _tables(out, partials, F[k], a_tab, b_tab, w1c, w1d)
        gsrc, gdst = _sc_gather(tsrc, tdst, src_idx, dst_idx)
        shift = _mlp(gsrc, gdst, ea_pad, w1e, b1r, W2, b2r, ar)
        partials = _sc_scatter(shift, dst_idx, zeros_nd)

    return _final(out, partials, F[4])


# speedup vs baseline: 6.0226x; 6.0226x over previous
import jax
import jax.numpy as jnp
from jax.experimental import pallas as pl
from jax.experimental.pallas import tpu as pltpu
from jax.experimental.pallas import tpu_sc as plsc

_N = 10000
_E = 160000
_EPAD = 163840
_DD = 128
_EH = 256
_GW = 128
_SW = 128
_ET = 2048
_NSEC = _EPAD // _ET
_NT = 1000
_NSUB = 16
_NACC = 10240
_ROWS_PER_SUB = _NACC // _NSUB

_F32 = jnp.float32


def _vmesh():
    return plsc.VectorSubcoreMesh(core_axis_name="core", subcore_axis_name="subcore")


def _pre_body(xs_ref, w1a_ref, w1b_ref, a_ref, b_ref):
    xs = xs_ref[...]
    a_ref[...] = jnp.dot(xs, w1a_ref[...], preferred_element_type=_F32)
    b_ref[...] = jnp.dot(xs, w1b_ref[...], preferred_element_type=_F32)


def _precompute(x_s, w1a, w1b):
    return pl.pallas_call(
        _pre_body,
        grid=(_N // _NT,),
        in_specs=[
            pl.BlockSpec((_NT, _DD), lambda i: (i, 0)),
            pl.BlockSpec((_DD, _EH), lambda i: (0, 0)),
            pl.BlockSpec((_DD, _EH), lambda i: (0, 0)),
        ],
        out_specs=[
            pl.BlockSpec((_NT, _EH), lambda i: (i, 0)),
            pl.BlockSpec((_NT, _EH), lambda i: (i, 0)),
        ],
        out_shape=[
            jax.ShapeDtypeStruct((_N, _EH), _F32),
            jax.ShapeDtypeStruct((_N, _EH), _F32),
        ],
    )(x_s, w1a, w1b)


def _tables_body(out_ref, p_ref, fk_ref, a_ref, b_ref, w1c_ref, w1d_ref,
                 onew_ref, tab_ref):
    out_new = out_ref[...] + jnp.dot(p_ref[0] + p_ref[1], fk_ref[...],
                                     preferred_element_type=_F32)
    onew_ref[...] = out_new
    s = a_ref[...] + jnp.dot(out_new, w1c_ref[...], preferred_element_type=_F32)
    d = b_ref[...] + jnp.dot(out_new, w1d_ref[...], preferred_element_type=_F32)
    tab_ref[0] = s[:, 0:128]
    tab_ref[1] = s[:, 128:256]
    tab_ref[2] = d[:, 0:128]
    tab_ref[3] = d[:, 128:256]
    tab_ref[4] = out_new


def _tables(out_prev, partials, fk, a_tab, b_tab, w1c, w1d):
    return pl.pallas_call(
        _tables_body,
        grid=(_N // _NT,),
        in_specs=[
            pl.BlockSpec((_NT, _DD), lambda i: (i, 0)),
            pl.BlockSpec((2, _NT, _DD), lambda i: (0, i, 0)),
            pl.BlockSpec((_DD, _DD), lambda i: (0, 0)),
            pl.BlockSpec((_NT, _EH), lambda i: (i, 0)),
            pl.BlockSpec((_NT, _EH), lambda i: (i, 0)),
            pl.BlockSpec((_DD, _EH), lambda i: (0, 0)),
            pl.BlockSpec((_DD, _EH), lambda i: (0, 0)),
        ],
        out_specs=[
            pl.BlockSpec((_NT, _DD), lambda i: (i, 0)),
            pl.BlockSpec((5, _NT, _DD), lambda i: (0, i, 0)),
        ],
        out_shape=[
            jax.ShapeDtypeStruct((_N, _DD), _F32),
            jax.ShapeDtypeStruct((5, _N, _DD), _F32),
        ],
    )(out_prev, partials, fk, a_tab, b_tab, w1c, w1d)


def _mlp_body(gs0_ref, gs1_ref, os_ref, gd0_ref, gd1_ref, od_ref, ea_ref,
              w1e_ref, b1_ref, w2a_ref, w2b_ref, b2_ref, a_ref, shift_ref):
    eb = jnp.dot(ea_ref[...], w1e_ref[...], preferred_element_type=_F32) \
        + b1_ref[...]
    h0 = gs0_ref[...] + gd0_ref[...] + eb[:, 0:128]
    h1 = gs1_ref[...] + gd1_ref[...] + eb[:, 128:256]
    alpha = a_ref[0, 0]
    h0 = jnp.where(h0 > 0.0, h0, alpha * h0)
    h1 = jnp.where(h1 > 0.0, h1, alpha * h1)
    w = (jnp.dot(h0, w2a_ref[...], preferred_element_type=_F32)
         + jnp.dot(h1, w2b_ref[...], preferred_element_type=_F32)
         + b2_ref[...])
    w = w / jnp.sqrt(jnp.sum(w * w, axis=1, keepdims=True))
    w = jnp.where(jnp.isnan(w), 0.0, w)
    os_ = os_ref[...]
    od = od_ref[...]
    ms = (jnp.sum(os_, axis=1, keepdims=True) != 0.0).astype(_F32)
    md = (jnp.sum(od, axis=1, keepdims=True) != 0.0).astype(_F32)
    shift_ref[...] = (od - os_) * w * jnp.maximum(ms, md)


def _mlp(gsrc, gdst, ea_pad, w1e, b1r, w2a, w2b, b2r, ar):
    return pl.pallas_call(
        _mlp_body,
        grid=(_NSEC,),
        in_specs=[
            pl.BlockSpec((_ET, _DD), lambda i: (i, 0)),
            pl.BlockSpec((_ET, _DD), lambda i: (i + _NSEC, 0)),
            pl.BlockSpec((_ET, _DD), lambda i: (i + 2 * _NSEC, 0)),
            pl.BlockSpec((_ET, _DD), lambda i: (i, 0)),
            pl.BlockSpec((_ET, _DD), lambda i: (i + _NSEC, 0)),
            pl.BlockSpec((_ET, _DD), lambda i: (i + 2 * _NSEC, 0)),
            pl.BlockSpec((_ET, 16), lambda i: (i, 0)),
            pl.BlockSpec((16, _EH), lambda i: (0, 0)),
            pl.BlockSpec((1, _EH), lambda i: (0, 0)),
            pl.BlockSpec((_DD, _DD), lambda i: (0, 0)),
            pl.BlockSpec((_DD, _DD), lambda i: (0, 0)),
            pl.BlockSpec((1, _DD), lambda i: (0, 0)),
            pl.BlockSpec((1, 1), lambda i: (0, 0)),
        ],
        out_specs=pl.BlockSpec((_ET, _DD), lambda i: (i, 0)),
        out_shape=jax.ShapeDtypeStruct((_EPAD, _DD), _F32),
    )(gsrc, gsrc, gsrc, gdst, gdst, gdst, ea_pad, w1e, b1r, w2a, w2b, b2r, ar)


def _final_body(out_ref, p_ref, fk_ref, onew_ref):
    onew_ref[...] = out_ref[...] + jnp.dot(p_ref[0] + p_ref[1], fk_ref[...],
                                           preferred_element_type=_F32)


def _final(out_prev, partials, fk):
    return pl.pallas_call(
        _final_body,
        grid=(_N // _NT,),
        in_specs=[
            pl.BlockSpec((_NT, _DD), lambda i: (i, 0)),
            pl.BlockSpec((2, _NT, _DD), lambda i: (0, i, 0)),
            pl.BlockSpec((_DD, _DD), lambda i: (0, 0)),
        ],
        out_specs=pl.BlockSpec((_NT, _DD), lambda i: (i, 0)),
        out_shape=jax.ShapeDtypeStruct((_N, _DD), _F32),
    )(out_prev, partials, fk)


def _sc_gather(table, idx, nrows):
    @pl.kernel(
        out_type=jax.ShapeDtypeStruct((nrows, _DD), _F32),
        mesh=_vmesh(),
    )
    def k(t_hbm, i_hbm, g_hbm):
        def body(i_v, g_v):
            pltpu.sync_copy(t_hbm.at[i_v.at[0]], g_v)

        pltpu.emit_pipeline(
            body,
            grid=(nrows // _GW,),
            in_specs=[
                pl.BlockSpec((1, _GW), lambda i: (0, i)),
            ],
            out_specs=[
                pl.BlockSpec((_GW, _DD), lambda i: (i, 0)),
            ],
            core_axis_name=("core", "subcore"),
            dimension_semantics=(pltpu.PARALLEL,),
        )(i_hbm, g_hbm)

    return k(table, idx)


def _sc_scatter(shift, dst_idx, zeros_nd):
    @pl.kernel(
        out_type=jax.ShapeDtypeStruct((2, _NACC, _DD), _F32),
        mesh=_vmesh(),
        scratch_types=[pltpu.VMEM_SHARED((_NACC, _DD), _F32)],
    )
    def k(shift_hbm, di_hbm, z_hbm, part_hbm, acc):
        core = jax.lax.axis_index("core")
        sid = jax.lax.axis_index("subcore")
        row0 = sid * _ROWS_PER_SUB
        pltpu.sync_copy(z_hbm.at[pl.ds(row0, _ROWS_PER_SUB)],
                        acc.at[pl.ds(row0, _ROWS_PER_SUB)])
        plsc.subcore_barrier()

        def body(sh_v, di_v):
            pltpu.sync_copy(sh_v, acc.at[di_v.at[0]], add=True)

        pltpu.emit_pipeline(
            body,
            grid=(_EPAD // _SW,),
            in_specs=[
                pl.BlockSpec((_SW, _DD), lambda i: (i, 0)),
                pl.BlockSpec((1, _SW), lambda i: (0, i)),
            ],
            out_specs=[],
            core_axis_name=("core", "subcore"),
            dimension_semantics=(pltpu.PARALLEL,),
        )(shift_hbm, di_hbm)
        plsc.subcore_barrier()
        pltpu.sync_copy(acc.at[pl.ds(row0, _ROWS_PER_SUB)],
                        part_hbm.at[core, pl.ds(row0, _ROWS_PER_SUB)])

    return k(shift, dst_idx, zeros_nd)


def kernel(x_s, x_t, edge_index, edge_attr, W1, b1, a, W2, b2, F):
    w1a = W1[0:128]
    w1b = W1[128:256]
    w1c = W1[256:384]
    w1d = W1[384:512]
    w1e = W1[512:528]
    b1r = b1.reshape(1, _EH)
    b2r = b2.reshape(1, _DD)
    ar = jnp.reshape(a, (1, 1))
    w2a = W2[0:128]
    w2b = W2[128:256]

    pad = _EPAD - _E
    src = jnp.pad(edge_index[0], (0, pad))
    dst = jnp.pad(edge_index[1], (0, pad))
    src_exp = jnp.concatenate([src, src + _N, src + 4 * _N]).reshape(1, -1)
    dst_exp = jnp.concatenate(
        [dst + 2 * _N, dst + 3 * _N, dst + 4 * _N]).reshape(1, -1)
    dst_idx = dst.reshape(1, _EPAD)
    ea_pad = jnp.pad(edge_attr, ((0, pad), (0, 0)))
    zeros_nd = jnp.zeros((_NACC, _DD), _F32)

    a_tab, b_tab = _precompute(x_s, w1a, w1b)

    out = jnp.zeros((_N, _DD), _F32)
    partials = jnp.concatenate(
        [jnp.pad(x_t, ((0, _NACC - _N), (0, 0))).reshape(1, _NACC, _DD),
         jnp.zeros((1, _NACC, _DD), _F32)], axis=0)

    for k in range(4):
        out, tab = _tables(out, partials, F[k], a_tab, b_tab, w1c, w1d)
        table = tab.reshape(5 * _N, _DD)
        gsrc = _sc_gather(table, src_exp, 3 * _EPAD)
        gdst = _sc_gather(table, dst_exp, 3 * _EPAD)
        shift = _mlp(gsrc, gdst, ea_pad, w1e, b1r, w2a, w2b, b2r, ar)
        partials = _sc_scatter(shift, dst_idx, zeros_nd)

    return _final(out, partials, F[4])
